# baseline (speedup 1.0000x reference)
import functools

import jax
import jax.numpy as jnp
from jax import lax
from jax.experimental import pallas as pl
from jax.experimental.pallas import tpu as pltpu
from jax.experimental.pallas import tpu_sc as plsc

N = 10000
E = 320000
D = 128

_NC = 2
_NS = 16
_NW = _NC * _NS
_EPW = E // _NW
_L = 16


def _nodeproj_body(x_ref, w_ref, o_ref):
    o_ref[...] = jnp.dot(x_ref[...], w_ref[...], preferred_element_type=jnp.float32)


def _node_projections(node_hidden, w12):
    return pl.pallas_call(
        _nodeproj_body,
        out_shape=jax.ShapeDtypeStruct((N, 2), jnp.float32),
    )(node_hidden, w12)


_sc_mesh = plsc.VectorSubcoreMesh(
    core_axis_name="c", subcore_axis_name="s", num_cores=_NC, num_subcores=_NS
)


@functools.partial(
    pl.kernel,
    out_type=jax.ShapeDtypeStruct((E,), jnp.float32),
    mesh=_sc_mesh,
    compiler_params=pltpu.CompilerParams(needs_layout_passes=False),
    scratch_types=[
        pltpu.VMEM((2 * N,), jnp.float32),
        pltpu.VMEM((_EPW,), jnp.int32),
        pltpu.VMEM((_EPW,), jnp.int32),
        pltpu.VMEM((_EPW,), jnp.float32),
    ],
)
def _sc_gather(st_hbm, src_hbm, dst_hbm, out_hbm, st_v, src_v, dst_v, g_v):
    wid = lax.axis_index("s") * _NC + lax.axis_index("c")
    base = wid * _EPW
    pltpu.sync_copy(st_hbm, st_v)
    pltpu.sync_copy(src_hbm.at[pl.ds(base, _EPW)], src_v)
    pltpu.sync_copy(dst_hbm.at[pl.ds(base, _EPW)], dst_v)

    def body(i, carry):
        sl = pl.ds(i * _L, _L)
        si = src_v[sl]
        di = dst_v[sl]
        g = plsc.load_gather(st_v, [si * 2]) + plsc.load_gather(st_v, [di * 2 + 1])
        g_v[sl] = g
        return carry

    lax.fori_loop(0, _EPW // _L, body, 0)
    pltpu.sync_copy(g_v, out_hbm.at[pl.ds(base, _EPW)])


_BE = 16000


_Q = 5
_M = E // _Q
_MB = 3200


def _decode_body(eh_ref, w_ref, b_ref, o_ref):
    x = eh_ref[...] * w_ref[...]
    hi = x.astype(jnp.bfloat16).astype(jnp.float32)
    lo = x - hi
    o_ref[...] = (jnp.sum(hi, axis=-1) + jnp.sum(lo, axis=-1)) + b_ref[0, 0]


def _edge_decode(edge_hidden3, w0row, b):
    return pl.pallas_call(
        _decode_body,
        grid=(_M // _MB,),
        in_specs=[
            pl.BlockSpec((_Q, _MB, D), lambda i: (0, i, 0)),
            pl.BlockSpec((1, 1, D), lambda i: (0, 0, 0)),
            pl.BlockSpec(memory_space=pltpu.SMEM),
        ],
        out_specs=pl.BlockSpec((_Q, _MB), lambda i: (0, i)),
        out_shape=jax.ShapeDtypeStruct((_Q, _M), jnp.float32),
    )(edge_hidden3, w0row, b)


def _combine_body(a_ref, g_ref, o_ref):
    o_ref[...] = a_ref[...] + g_ref[...]


def _combine(edgedot, g):
    return pl.pallas_call(
        _combine_body,
        out_shape=jax.ShapeDtypeStruct((_Q, _M), jnp.float32),
    )(edgedot, g)


def kernel(node_hidden, edge_hidden, edge_index, W, b):
    src = edge_index[0].astype(jnp.int32)
    dst = edge_index[1].astype(jnp.int32)
    w0 = W[:D]
    w12 = jnp.concatenate([W[D : 2 * D], W[2 * D :]], axis=1)

    st = _node_projections(node_hidden, w12).reshape(2 * N)
    g = _sc_gather(st, src, dst).reshape(_Q, _M)
    eh3 = edge_hidden.reshape(_Q, _M, D)
    edgedot = _edge_decode(eh3, w0.reshape(1, 1, D), b.reshape(1, 1))
    return _combine(edgedot, g).reshape(E, 1)


# speedup vs baseline: 1.3033x; 1.3033x over previous
import functools

import jax
import jax.numpy as jnp
from jax import lax
from jax.experimental import pallas as pl
from jax.experimental.pallas import tpu as pltpu
from jax.experimental.pallas import tpu_sc as plsc

N = 10000
E = 320000
D = 128

_NC = 2
_NS = 16
_NW = _NC * _NS
_EPW = E // _NW
_L = 16


def _nodeproj_body(x_ref, w_ref, o_ref):
    o_ref[...] = jnp.dot(x_ref[...], w_ref[...], preferred_element_type=jnp.float32)


def _node_projections(node_hidden, w12):
    return pl.pallas_call(
        _nodeproj_body,
        out_shape=jax.ShapeDtypeStruct((N, 2), jnp.float32),
    )(node_hidden, w12)


_sc_mesh = plsc.VectorSubcoreMesh(
    core_axis_name="c", subcore_axis_name="s", num_cores=_NC, num_subcores=_NS
)


@functools.partial(
    pl.kernel,
    out_type=jax.ShapeDtypeStruct((E,), jnp.float32),
    mesh=_sc_mesh,
    compiler_params=pltpu.CompilerParams(needs_layout_passes=False),
    scratch_types=[
        pltpu.VMEM((2 * N,), jnp.float32),
        pltpu.VMEM((_EPW,), jnp.int32),
        pltpu.VMEM((_EPW,), jnp.int32),
        pltpu.VMEM((_EPW,), jnp.float32),
    ],
)
def _sc_gather(st_hbm, src_hbm, dst_hbm, out_hbm, st_v, src_v, dst_v, g_v):
    wid = lax.axis_index("s") * _NC + lax.axis_index("c")
    base = wid * _EPW
    pltpu.sync_copy(st_hbm, st_v)
    pltpu.sync_copy(src_hbm.at[pl.ds(base, _EPW)], src_v)
    pltpu.sync_copy(dst_hbm.at[pl.ds(base, _EPW)], dst_v)

    def body(i, carry):
        sl = pl.ds(i * _L, _L)
        si = src_v[sl]
        di = dst_v[sl]
        g = plsc.load_gather(st_v, [si * 2]) + plsc.load_gather(st_v, [di * 2 + 1])
        g_v[sl] = g
        return carry

    lax.fori_loop(0, _EPW // _L, body, 0)
    pltpu.sync_copy(g_v, out_hbm.at[pl.ds(base, _EPW)])


_BE = 16000


_Q = 5
_M = E // _Q
_MB = 6400


def _decode_body(eh_ref, w_ref, b_ref, o_ref):
    o_ref[...] = jnp.sum(eh_ref[...] * w_ref[...], axis=-1) + b_ref[0, 0]


def _edge_decode(edge_hidden3, w0row, b):
    return pl.pallas_call(
        _decode_body,
        grid=(_M // _MB,),
        in_specs=[
            pl.BlockSpec((_Q, _MB, D), lambda i: (0, i, 0)),
            pl.BlockSpec((1, 1, D), lambda i: (0, 0, 0)),
            pl.BlockSpec(memory_space=pltpu.SMEM),
        ],
        out_specs=pl.BlockSpec((_Q, _MB), lambda i: (0, i)),
        out_shape=jax.ShapeDtypeStruct((_Q, _M), jnp.float32),
    )(edge_hidden3, w0row, b)


def _combine_body(a_ref, g_ref, o_ref):
    o_ref[...] = a_ref[...] + g_ref[...]


def _combine(edgedot, g):
    return pl.pallas_call(
        _combine_body,
        out_shape=jax.ShapeDtypeStruct((_Q, _M), jnp.float32),
    )(edgedot, g)


def kernel(node_hidden, edge_hidden, edge_index, W, b):
    src = edge_index[0].astype(jnp.int32)
    dst = edge_index[1].astype(jnp.int32)
    w0 = W[:D]
    w12 = jnp.concatenate([W[D : 2 * D], W[2 * D :]], axis=1)

    st = _node_projections(node_hidden, w12).reshape(2 * N)
    g = _sc_gather(st, src, dst).reshape(_Q, _M)
    eh3 = edge_hidden.reshape(_Q, _M, D)
    edgedot = _edge_decode(eh3, w0.reshape(1, 1, D), b.reshape(1, 1))
    return _combine(edgedot, g).reshape(E, 1)


# speedup vs baseline: 1.3366x; 1.0255x over previous
import functools

import jax
import jax.numpy as jnp
from jax import lax
from jax.experimental import pallas as pl
from jax.experimental.pallas import tpu as pltpu
from jax.experimental.pallas import tpu_sc as plsc

N = 10000
E = 320000
D = 128

_NC = 2
_NS = 16
_NW = _NC * _NS
_EPW = E // _NW
_L = 16

_NPT = 640
_NH = 320
_sc_mesh = plsc.VectorSubcoreMesh(
    core_axis_name="c", subcore_axis_name="s", num_cores=_NC, num_subcores=_NS
)


@functools.partial(
    pl.kernel,
    out_type=jax.ShapeDtypeStruct((E,), jnp.float32),
    mesh=_sc_mesh,
    compiler_params=pltpu.CompilerParams(needs_layout_passes=False),
    scratch_types=[
        pltpu.VMEM((_NH, D), jnp.float32),
        pltpu.VMEM((2 * D,), jnp.float32),
        pltpu.VMEM((2 * _NPT,), jnp.float32),
        pltpu.VMEM((2 * N,), jnp.float32),
        pltpu.VMEM((_EPW,), jnp.int32),
        pltpu.VMEM((_EPW,), jnp.int32),
        pltpu.VMEM((_EPW,), jnp.float32),
        pltpu.VMEM_SHARED((2 * N,), jnp.float32),
    ],
)
def _sc_gather(node_hbm, w12_hbm, src_hbm, dst_hbm, out_hbm,
               nodes_v, w_v, stl_v, st_v, src_v, dst_v, g_v, st_sh):
    tid = lax.axis_index("s")
    wid = tid * _NC + lax.axis_index("c")
    nbase = jnp.minimum(tid * _NPT, N - _NPT)
    ebase = wid * _EPW

    pltpu.sync_copy(w12_hbm, w_v)
    pltpu.sync_copy(src_hbm.at[pl.ds(ebase, _EPW)], src_v)
    pltpu.sync_copy(dst_hbm.at[pl.ds(ebase, _EPW)], dst_v)

    lanes = lax.iota(jnp.int32, _L)

    for h in range(_NPT // _NH):
        pltpu.sync_copy(node_hbm.at[pl.ds(nbase + h * _NH, _NH)], nodes_v)

        def chunk(c, carry):
            rows = c * _L + lanes

            def feat(d, accs):
                acc_s, acc_t = accs
                dv = jnp.zeros((_L,), jnp.int32) + d
                col = plsc.load_gather(nodes_v, [rows, dv])
                w1d = plsc.load_gather(w_v, [dv])
                w2d = plsc.load_gather(w_v, [dv + D])
                return (acc_s + col * w1d, acc_t + col * w2d)

            z = jnp.zeros((_L,), jnp.float32)
            acc_s, acc_t = lax.fori_loop(0, D, feat, (z, z), unroll=4)
            loc = (h * _NH + c * _L) + lanes
            plsc.store_scatter(stl_v, [2 * loc], acc_s)
            plsc.store_scatter(stl_v, [2 * loc + 1], acc_t)
            return carry

        lax.fori_loop(0, _NH // _L, chunk, 0)

    pltpu.sync_copy(stl_v, st_sh.at[pl.ds(2 * nbase, 2 * _NPT)])
    plsc.subcore_barrier()
    pltpu.sync_copy(st_sh, st_v)

    def body(i, carry):
        sl = pl.ds(i * _L, _L)
        si = src_v[sl]
        di = dst_v[sl]
        g = plsc.load_gather(st_v, [si * 2]) + plsc.load_gather(st_v, [di * 2 + 1])
        g_v[sl] = g
        return carry

    lax.fori_loop(0, _EPW // _L, body, 0, unroll=4)
    pltpu.sync_copy(g_v, out_hbm.at[pl.ds(ebase, _EPW)])


_Q = 5
_M = E // _Q
_MB = 6400


def _decode_body(eh_ref, w_ref, b_ref, o_ref):
    o_ref[...] = jnp.sum(eh_ref[...] * w_ref[...], axis=-1) + b_ref[0, 0]


def _edge_decode(edge_hidden3, w0row, b):
    return pl.pallas_call(
        _decode_body,
        grid=(_M // _MB,),
        in_specs=[
            pl.BlockSpec((_Q, _MB, D), lambda i: (0, i, 0)),
            pl.BlockSpec((1, 1, D), lambda i: (0, 0, 0)),
            pl.BlockSpec(memory_space=pltpu.SMEM),
        ],
        out_specs=pl.BlockSpec((_Q, _MB), lambda i: (0, i)),
        out_shape=jax.ShapeDtypeStruct((_Q, _M), jnp.float32),
    )(edge_hidden3, w0row, b)


def _combine_body(a_ref, g_ref, o_ref):
    o_ref[...] = a_ref[...] + g_ref[...]


def _combine(edgedot, g):
    return pl.pallas_call(
        _combine_body,
        out_shape=jax.ShapeDtypeStruct((_Q, _M), jnp.float32),
    )(edgedot, g)


def kernel(node_hidden, edge_hidden, edge_index, W, b):
    src = edge_index[0].astype(jnp.int32)
    dst = edge_index[1].astype(jnp.int32)
    w0 = W[:D]
    w12 = W[D:].reshape(2 * D)

    g = _sc_gather(node_hidden, w12, src, dst).reshape(_Q, _M)
    eh3 = edge_hidden.reshape(_Q, _M, D)
    edgedot = _edge_decode(eh3, w0.reshape(1, 1, D), b.reshape(1, 1))
    return _combine(edgedot, g).reshape(E, 1)


# speedup vs baseline: 1.3400x; 1.0026x over previous
import functools

import jax
import jax.numpy as jnp
from jax import lax
from jax.experimental import pallas as pl
from jax.experimental.pallas import tpu as pltpu
from jax.experimental.pallas import tpu_sc as plsc

N = 10000
E = 320000
D = 128

_NC = 2
_NS = 16
_NW = _NC * _NS
_EPW = E // _NW
_L = 16

_NPT = 640
_NH = 320
_sc_mesh = plsc.VectorSubcoreMesh(
    core_axis_name="c", subcore_axis_name="s", num_cores=_NC, num_subcores=_NS
)


@functools.partial(
    pl.kernel,
    out_type=jax.ShapeDtypeStruct((E,), jnp.float32),
    mesh=_sc_mesh,
    compiler_params=pltpu.CompilerParams(needs_layout_passes=False),
    scratch_types=[
        pltpu.VMEM((_NH, D), jnp.float32),
        pltpu.VMEM((2 * D,), jnp.float32),
        pltpu.VMEM((2 * _NPT,), jnp.float32),
        pltpu.VMEM((2 * N,), jnp.float32),
        pltpu.VMEM((_EPW,), jnp.int32),
        pltpu.VMEM((_EPW,), jnp.int32),
        pltpu.VMEM((_EPW,), jnp.float32),
        pltpu.VMEM_SHARED((2 * N,), jnp.float32),
    ],
)
def _sc_gather(node_hbm, w12_hbm, src_hbm, dst_hbm, out_hbm,
               nodes_v, w_v, stl_v, st_v, src_v, dst_v, g_v, st_sh):
    tid = lax.axis_index("s")
    wid = tid * _NC + lax.axis_index("c")
    nbase = jnp.minimum(tid * _NPT, N - _NPT)
    ebase = wid * _EPW

    pltpu.sync_copy(w12_hbm, w_v)
    pltpu.sync_copy(src_hbm.at[pl.ds(ebase, _EPW)], src_v)
    pltpu.sync_copy(dst_hbm.at[pl.ds(ebase, _EPW)], dst_v)

    lanes = lax.iota(jnp.int32, _L)

    for h in range(_NPT // _NH):
        pltpu.sync_copy(node_hbm.at[pl.ds(nbase + h * _NH, _NH)], nodes_v)

        def chunk(c, carry):
            rows = c * _L + lanes

            def feat(d, accs):
                acc_s, acc_t = accs
                dv = jnp.zeros((_L,), jnp.int32) + d
                col = plsc.load_gather(nodes_v, [rows, dv])
                w1d = plsc.load_gather(w_v, [dv])
                w2d = plsc.load_gather(w_v, [dv + D])
                return (acc_s + col * w1d, acc_t + col * w2d)

            z = jnp.zeros((_L,), jnp.float32)
            acc_s, acc_t = lax.fori_loop(0, D, feat, (z, z), unroll=4)
            loc = (h * _NH + c * _L) + lanes
            plsc.store_scatter(stl_v, [2 * loc], acc_s)
            plsc.store_scatter(stl_v, [2 * loc + 1], acc_t)
            return carry

        lax.fori_loop(0, _NH // _L, chunk, 0)

    pltpu.sync_copy(stl_v, st_sh.at[pl.ds(2 * nbase, 2 * _NPT)])
    plsc.subcore_barrier()
    pltpu.sync_copy(st_sh, st_v)

    def body(i, carry):
        sl = pl.ds(i * _L, _L)
        si = src_v[sl]
        di = dst_v[sl]
        g = plsc.load_gather(st_v, [si * 2]) + plsc.load_gather(st_v, [di * 2 + 1])
        g_v[sl] = g
        return carry

    lax.fori_loop(0, _EPW // _L, body, 0, unroll=4)
    pltpu.sync_copy(g_v, out_hbm.at[pl.ds(ebase, _EPW)])


_Q = 5
_M = E // _Q
_MB = 12800


def _decode_body(eh_ref, w_ref, b_ref, o_ref):
    o_ref[...] = jnp.sum(eh_ref[...] * w_ref[...], axis=-1) + b_ref[0, 0]


def _edge_decode(edge_hidden3, w0row, b):
    return pl.pallas_call(
        _decode_body,
        grid=(_M // _MB,),
        compiler_params=pltpu.CompilerParams(vmem_limit_bytes=100 * 1024 * 1024),
        in_specs=[
            pl.BlockSpec((_Q, _MB, D), lambda i: (0, i, 0)),
            pl.BlockSpec((1, 1, D), lambda i: (0, 0, 0)),
            pl.BlockSpec(memory_space=pltpu.SMEM),
        ],
        out_specs=pl.BlockSpec((_Q, _MB), lambda i: (0, i)),
        out_shape=jax.ShapeDtypeStruct((_Q, _M), jnp.float32),
    )(edge_hidden3, w0row, b)


def _combine_body(a_ref, g_ref, o_ref):
    o_ref[...] = a_ref[...] + g_ref[...]


def _combine(edgedot, g):
    return pl.pallas_call(
        _combine_body,
        out_shape=jax.ShapeDtypeStruct((_Q, _M), jnp.float32),
    )(edgedot, g)


def kernel(node_hidden, edge_hidden, edge_index, W, b):
    src = edge_index[0].astype(jnp.int32)
    dst = edge_index[1].astype(jnp.int32)
    w0 = W[:D]
    w12 = W[D:].reshape(2 * D)

    g = _sc_gather(node_hidden, w12, src, dst).reshape(_Q, _M)
    eh3 = edge_hidden.reshape(_Q, _M, D)
    edgedot = _edge_decode(eh3, w0.reshape(1, 1, D), b.reshape(1, 1))
    return _combine(edgedot, g).reshape(E, 1)
